# Initial kernel scaffold; baseline (speedup 1.0000x reference)
import jax
import jax.numpy as jnp
from jax.experimental import pallas as pl


def kernel(x, edge_index, edge_attr, Wpn, bpn, Wpe, bpe, W1_0, b1_0, W2_0, b2_0, W1_1, b1_1, W2_1, b2_1, W1_2, b1_2, W2_2, b2_2, Wsp, bsp, prelu_a):
    raise NotImplementedError("write your pallas kernel here")


# speedup vs baseline: 3.3577x; 3.3577x over previous
import functools

import jax
import jax.numpy as jnp
from jax import lax
from jax.experimental import pallas as pl
from jax.experimental.pallas import tpu as pltpu
from jax.experimental.pallas import tpu_sc as plsc

N = 50000
E = 800000
DIN = 128
DE = 16
H = 64
HH = 32
DOUT = 1024

_F32 = jnp.float32


BM_N = 5000
BM_E = 10000


def _proj_body(x_ref, w_ref, b_ref, lo_ref, hi_ref, *, relu):
    y = jnp.dot(x_ref[...], w_ref[...], preferred_element_type=_F32) + b_ref[...]
    if relu:
        y = jnp.maximum(y, 0.0)
    lo_ref[...] = y[:, :HH]
    hi_ref[...] = y[:, HH:]


def _proj(x, w, b, bm, relu):
    rows, k = x.shape
    return pl.pallas_call(
        functools.partial(_proj_body, relu=relu),
        grid=(rows // bm,),
        in_specs=[
            pl.BlockSpec((bm, k), lambda i: (i, 0)),
            pl.BlockSpec((k, H), lambda i: (0, 0)),
            pl.BlockSpec((1, H), lambda i: (0, 0)),
        ],
        out_specs=[
            pl.BlockSpec((bm, HH), lambda i: (i, 0)),
            pl.BlockSpec((bm, HH), lambda i: (i, 0)),
        ],
        out_shape=[
            jax.ShapeDtypeStruct((rows, HH), _F32),
            jax.ShapeDtypeStruct((rows, HH), _F32),
        ],
    )(x, w, b.reshape(1, H))


def _mlp_body(hlo_ref, hhi_ref, alo_ref, ahi_ref, w1_ref, b1_ref, w2_ref, b2_ref,
              olo_ref, ohi_ref):
    zlo = hlo_ref[...] + alo_ref[...]
    zhi = hhi_ref[...] + ahi_ref[...]
    w1 = w1_ref[...]
    t = (jnp.dot(zlo, w1[:HH, :], preferred_element_type=_F32)
         + jnp.dot(zhi, w1[HH:, :], preferred_element_type=_F32) + b1_ref[...])
    t = jnp.maximum(t, 0.0)
    u = jnp.dot(t, w2_ref[...], preferred_element_type=_F32) + b2_ref[...]
    u = jnp.maximum(u, 0.0)
    olo_ref[...] = u[:, :HH]
    ohi_ref[...] = u[:, HH:]


def _mlp(hlo, hhi, alo, ahi, w1, b1, w2, b2):
    return pl.pallas_call(
        _mlp_body,
        grid=(N // BM_N,),
        in_specs=[
            pl.BlockSpec((BM_N, HH), lambda i: (i, 0)),
            pl.BlockSpec((BM_N, HH), lambda i: (i, 0)),
            pl.BlockSpec((BM_N, HH), lambda i: (i, 0)),
            pl.BlockSpec((BM_N, HH), lambda i: (i, 0)),
            pl.BlockSpec((H, H), lambda i: (0, 0)),
            pl.BlockSpec((1, H), lambda i: (0, 0)),
            pl.BlockSpec((H, H), lambda i: (0, 0)),
            pl.BlockSpec((1, H), lambda i: (0, 0)),
        ],
        out_specs=[
            pl.BlockSpec((BM_N, HH), lambda i: (i, 0)),
            pl.BlockSpec((BM_N, HH), lambda i: (i, 0)),
        ],
        out_shape=[
            jax.ShapeDtypeStruct((N, HH), _F32),
            jax.ShapeDtypeStruct((N, HH), _F32),
        ],
    )(hlo, hhi, alo, ahi, w1, b1.reshape(1, H), w2, b2.reshape(1, H))


def _final_body(hlo_ref, hhi_ref, alo_ref, ahi_ref, w1_ref, b1_ref, w2_ref, b2_ref,
                wsp_ref, bsp_ref, pa_ref, out_ref, acc_ref):
    i = pl.program_id(0)
    zlo = hlo_ref[...] + alo_ref[...]
    zhi = hhi_ref[...] + ahi_ref[...]
    w1 = w1_ref[...]
    t = (jnp.dot(zlo, w1[:HH, :], preferred_element_type=_F32)
         + jnp.dot(zhi, w1[HH:, :], preferred_element_type=_F32) + b1_ref[...])
    t = jnp.maximum(t, 0.0)
    u = jnp.dot(t, w2_ref[...], preferred_element_type=_F32) + b2_ref[...]
    part = jnp.sum(u, axis=0, keepdims=True)

    @pl.when(i == 0)
    def _():
        acc_ref[...] = part

    @pl.when(i > 0)
    def _():
        acc_ref[...] = acc_ref[...] + part

    @pl.when(i == pl.num_programs(0) - 1)
    def _():
        ro = acc_ref[...] * _F32(1.0 / N)
        sv = jnp.dot(ro, wsp_ref[...], preferred_element_type=_F32) + bsp_ref[...]
        out_ref[...] = jnp.where(sv >= 0.0, sv, pa_ref[...] * sv)


def _final(hlo, hhi, alo, ahi, w1, b1, w2, b2, wsp, bsp, prelu_a):
    return pl.pallas_call(
        _final_body,
        grid=(N // BM_N,),
        in_specs=[
            pl.BlockSpec((BM_N, HH), lambda i: (i, 0)),
            pl.BlockSpec((BM_N, HH), lambda i: (i, 0)),
            pl.BlockSpec((BM_N, HH), lambda i: (i, 0)),
            pl.BlockSpec((BM_N, HH), lambda i: (i, 0)),
            pl.BlockSpec((H, H), lambda i: (0, 0)),
            pl.BlockSpec((1, H), lambda i: (0, 0)),
            pl.BlockSpec((H, H), lambda i: (0, 0)),
            pl.BlockSpec((1, H), lambda i: (0, 0)),
            pl.BlockSpec((H, DOUT), lambda i: (0, 0)),
            pl.BlockSpec((1, DOUT), lambda i: (0, 0)),
            pl.BlockSpec((1, 1), lambda i: (0, 0)),
        ],
        out_specs=pl.BlockSpec((1, DOUT), lambda i: (0, 0)),
        out_shape=jax.ShapeDtypeStruct((1, DOUT), _F32),
        scratch_shapes=[pltpu.VMEM((1, H), _F32)],
    )(hlo, hhi, alo, ahi, w1, b1.reshape(1, H), w2, b2.reshape(1, H),
      wsp, bsp.reshape(1, DOUT), prelu_a.reshape(1, 1))


NSUB = 16
TE = E // NSUB
G = 2
GCH = 125
SCH = G * GCH
NSC = TE // SCH
NTILE = 3128
NTILE_LAST = N - 15 * NTILE

_mesh = plsc.VectorSubcoreMesh(core_axis_name="c", subcore_axis_name="s",
                               num_cores=2, num_subcores=NSUB)


@functools.partial(
    pl.kernel,
    out_type=[jax.ShapeDtypeStruct((N, HH), _F32),
              jax.ShapeDtypeStruct((N, HH), _F32)],
    mesh=_mesh,
    scratch_types=[
        pltpu.VMEM_SHARED((N, HH), _F32),
        pltpu.VMEM((2, G, GCH), jnp.int32),
        pltpu.VMEM((2, G, GCH), jnp.int32),
        pltpu.VMEM((SCH, HH), _F32),
        pltpu.VMEM((2, SCH, HH), _F32),
        pltpu.SemaphoreType.DMA,
        pltpu.SemaphoreType.DMA,
        pltpu.SemaphoreType.DMA,
    ],
    compiler_params=pltpu.CompilerParams(use_tc_tiling_on_sc=False),
)
def _sc_message(hlo, hhi, elo, ehi, src3d, dst3d, olo, ohi,
                aggr, srcb, dstb, rows, ebuf, lsem0, lsem1, gsem):
    c = lax.axis_index("c")
    s = lax.axis_index("s")

    def run(h_ref, e_ref, out_ref):
        def zero_body(j, carry):
            z = jnp.zeros((16,), _F32)
            ebuf[0, j, pl.ds(0, 16)] = z
            ebuf[0, j, pl.ds(16, 16)] = z
            return carry
        lax.fori_loop(0, SCH, zero_body, 0)
        base = s * NTILE
        nfull = NTILE // SCH
        for k in range(nfull):
            pltpu.sync_copy(ebuf.at[0], aggr.at[pl.ds(base + k * SCH, SCH)])

        @pl.when(s < NSUB - 1)
        def _():
            pltpu.sync_copy(ebuf.at[0, pl.ds(0, NTILE - nfull * SCH)],
                            aggr.at[pl.ds(base + nfull * SCH, NTILE - nfull * SCH)])

        @pl.when(s == NSUB - 1)
        def _():
            pltpu.sync_copy(ebuf.at[0, pl.ds(0, NTILE_LAST - nfull * SCH)],
                            aggr.at[pl.ds(base + nfull * SCH, NTILE_LAST - nfull * SCH)])

        plsc.subcore_barrier()

        ebase = s * TE
        rbase = s * NSC

        def lin_views(t, b):
            off_e = ebase + t * SCH
            return [
                (src3d.at[rbase + t], srcb.at[b]),
                (dst3d.at[rbase + t], dstb.at[b]),
                (e_ref.at[pl.ds(off_e, SCH)], ebuf.at[b]),
            ]

        def issue_linear(t, b, sem):
            for sv, dv in lin_views(t, b):
                pltpu.async_copy(sv, dv, sem)

        def drain_linear(t, b, sem):
            for sv, dv in lin_views(t, b):
                pltpu.make_async_copy(sv, dv, sem).wait()

        issue_linear(0, 0, lsem0)
        issue_linear(1, 1, lsem1)

        def step(t, b, sem):
            drain_linear(t, b, sem)
            descs = [
                pltpu.async_copy(h_ref.at[srcb.at[b, g]],
                                 rows.at[pl.ds(g * GCH, GCH)], gsem)
                for g in range(G)
            ]
            for d in descs:
                d.wait()

            def addrelu(j, carry):
                for q in range(2):
                    sl = pl.ds(q * 16, 16)
                    rows[j, sl] = jnp.maximum(rows[j, sl] + ebuf[b, j, sl], 0.0)
                return carry
            lax.fori_loop(0, SCH, addrelu, 0)

            for g in range(G):
                pltpu.sync_copy(rows.at[pl.ds(g * GCH, GCH)],
                                aggr.at[dstb.at[b, g]], add=True)

            @pl.when(t + 2 < NSC)
            def _():
                issue_linear(t + 2, b, sem)

        def pair(i, carry):
            step(2 * i, 0, lsem0)
            step(2 * i + 1, 1, lsem1)
            return carry
        lax.fori_loop(0, NSC // 2, pair, 0)

        plsc.subcore_barrier()

        @pl.when(s < NSUB - 1)
        def _():
            pltpu.sync_copy(aggr.at[pl.ds(base, NTILE)],
                            out_ref.at[pl.ds(base, NTILE)])

        @pl.when(s == NSUB - 1)
        def _():
            pltpu.sync_copy(aggr.at[pl.ds(base, NTILE_LAST)],
                            out_ref.at[pl.ds(base, NTILE_LAST)])

    @pl.when(c == 0)
    def _():
        run(hlo, elo, olo)

    @pl.when(c == 1)
    def _():
        run(hhi, ehi, ohi)


def kernel(x, edge_index, edge_attr, Wpn, bpn, Wpe, bpe, W1_0, b1_0, W2_0, b2_0,
           W1_1, b1_1, W2_1, b2_1, W1_2, b1_2, W2_2, b2_2, Wsp, bsp, prelu_a):
    src3d = edge_index[0].reshape(E // SCH, G, GCH)
    dst3d = edge_index[1].reshape(E // SCH, G, GCH)
    hlo, hhi = _proj(x, Wpn, bpn, BM_N, relu=True)
    elo, ehi = _proj(edge_attr, Wpe, bpe, BM_E, relu=False)
    for (w1, b1, w2, b2) in ((W1_0, b1_0, W2_0, b2_0), (W1_1, b1_1, W2_1, b2_1)):
        alo, ahi = _sc_message(hlo, hhi, elo, ehi, src3d, dst3d)
        hlo, hhi = _mlp(hlo, hhi, alo, ahi, w1, b1, w2, b2)
    alo, ahi = _sc_message(hlo, hhi, elo, ehi, src3d, dst3d)
    return _final(hlo, hhi, alo, ahi, W1_2, b1_2, W2_2, b2_2, Wsp, bsp, prelu_a)


# speedup vs baseline: 4.7043x; 1.4011x over previous
import functools

import jax
import jax.numpy as jnp
from jax import lax
from jax.experimental import pallas as pl
from jax.experimental.pallas import tpu as pltpu
from jax.experimental.pallas import tpu_sc as plsc

N = 50000
E = 800000
DIN = 128
DE = 16
H = 64
HH = 32
DOUT = 1024

_F32 = jnp.float32


BM_N = 5000
BM_E = 10000


def _proj_body(x_ref, w_ref, b_ref, lo_ref, hi_ref, *, relu):
    y = jnp.dot(x_ref[...], w_ref[...], preferred_element_type=_F32) + b_ref[...]
    if relu:
        y = jnp.maximum(y, 0.0)
    lo_ref[...] = y[:, :HH]
    hi_ref[...] = y[:, HH:]


def _proj(x, w, b, bm, relu):
    rows, k = x.shape
    return pl.pallas_call(
        functools.partial(_proj_body, relu=relu),
        grid=(rows // bm,),
        in_specs=[
            pl.BlockSpec((bm, k), lambda i: (i, 0)),
            pl.BlockSpec((k, H), lambda i: (0, 0)),
            pl.BlockSpec((1, H), lambda i: (0, 0)),
        ],
        out_specs=[
            pl.BlockSpec((bm, HH), lambda i: (i, 0)),
            pl.BlockSpec((bm, HH), lambda i: (i, 0)),
        ],
        out_shape=[
            jax.ShapeDtypeStruct((rows, HH), _F32),
            jax.ShapeDtypeStruct((rows, HH), _F32),
        ],
    )(x, w, b.reshape(1, H))


EPACK = 8
EROWS = E // EPACK
BM_E8 = 5000


def _proj_e_body(a_ref, wlo_ref, whi_ref, blo_ref, bhi_ref, lo_ref, hi_ref):
    a = a_ref[...]
    lo_ref[...] = jnp.dot(a, wlo_ref[...], preferred_element_type=_F32) + blo_ref[...]
    hi_ref[...] = jnp.dot(a, whi_ref[...], preferred_element_type=_F32) + bhi_ref[...]


def _proj_e(ea8, w8lo, w8hi, b8lo, b8hi):
    kk = EPACK * DE
    dd = EPACK * HH
    return pl.pallas_call(
        _proj_e_body,
        grid=(EROWS // BM_E8,),
        in_specs=[
            pl.BlockSpec((BM_E8, kk), lambda i: (i, 0)),
            pl.BlockSpec((kk, dd), lambda i: (0, 0)),
            pl.BlockSpec((kk, dd), lambda i: (0, 0)),
            pl.BlockSpec((1, dd), lambda i: (0, 0)),
            pl.BlockSpec((1, dd), lambda i: (0, 0)),
        ],
        out_specs=[
            pl.BlockSpec((BM_E8, dd), lambda i: (i, 0)),
            pl.BlockSpec((BM_E8, dd), lambda i: (i, 0)),
        ],
        out_shape=[
            jax.ShapeDtypeStruct((EROWS, dd), _F32),
            jax.ShapeDtypeStruct((EROWS, dd), _F32),
        ],
    )(ea8, w8lo, w8hi, b8lo, b8hi)


def _mlp_body(hlo_ref, hhi_ref, alo_ref, ahi_ref, w1_ref, b1_ref, w2_ref, b2_ref,
              olo_ref, ohi_ref):
    zlo = hlo_ref[...] + alo_ref[...]
    zhi = hhi_ref[...] + ahi_ref[...]
    w1 = w1_ref[...]
    t = (jnp.dot(zlo, w1[:HH, :], preferred_element_type=_F32)
         + jnp.dot(zhi, w1[HH:, :], preferred_element_type=_F32) + b1_ref[...])
    t = jnp.maximum(t, 0.0)
    u = jnp.dot(t, w2_ref[...], preferred_element_type=_F32) + b2_ref[...]
    u = jnp.maximum(u, 0.0)
    olo_ref[...] = u[:, :HH]
    ohi_ref[...] = u[:, HH:]


def _mlp(hlo, hhi, alo, ahi, w1, b1, w2, b2):
    return pl.pallas_call(
        _mlp_body,
        grid=(N // BM_N,),
        in_specs=[
            pl.BlockSpec((BM_N, HH), lambda i: (i, 0)),
            pl.BlockSpec((BM_N, HH), lambda i: (i, 0)),
            pl.BlockSpec((BM_N, HH), lambda i: (i, 0)),
            pl.BlockSpec((BM_N, HH), lambda i: (i, 0)),
            pl.BlockSpec((H, H), lambda i: (0, 0)),
            pl.BlockSpec((1, H), lambda i: (0, 0)),
            pl.BlockSpec((H, H), lambda i: (0, 0)),
            pl.BlockSpec((1, H), lambda i: (0, 0)),
        ],
        out_specs=[
            pl.BlockSpec((BM_N, HH), lambda i: (i, 0)),
            pl.BlockSpec((BM_N, HH), lambda i: (i, 0)),
        ],
        out_shape=[
            jax.ShapeDtypeStruct((N, HH), _F32),
            jax.ShapeDtypeStruct((N, HH), _F32),
        ],
    )(hlo, hhi, alo, ahi, w1, b1.reshape(1, H), w2, b2.reshape(1, H))


def _final_body(hlo_ref, hhi_ref, alo_ref, ahi_ref, w1_ref, b1_ref, w2_ref, b2_ref,
                wsp_ref, bsp_ref, pa_ref, out_ref, acc_ref):
    i = pl.program_id(0)
    zlo = hlo_ref[...] + alo_ref[...]
    zhi = hhi_ref[...] + ahi_ref[...]
    w1 = w1_ref[...]
    t = (jnp.dot(zlo, w1[:HH, :], preferred_element_type=_F32)
         + jnp.dot(zhi, w1[HH:, :], preferred_element_type=_F32) + b1_ref[...])
    t = jnp.maximum(t, 0.0)
    u = jnp.dot(t, w2_ref[...], preferred_element_type=_F32) + b2_ref[...]
    part = jnp.sum(u, axis=0, keepdims=True)

    @pl.when(i == 0)
    def _():
        acc_ref[...] = part

    @pl.when(i > 0)
    def _():
        acc_ref[...] = acc_ref[...] + part

    @pl.when(i == pl.num_programs(0) - 1)
    def _():
        ro = acc_ref[...] * _F32(1.0 / N)
        sv = jnp.dot(ro, wsp_ref[...], preferred_element_type=_F32) + bsp_ref[...]
        out_ref[...] = jnp.where(sv >= 0.0, sv, pa_ref[...] * sv)


def _final(hlo, hhi, alo, ahi, w1, b1, w2, b2, wsp, bsp, prelu_a):
    return pl.pallas_call(
        _final_body,
        grid=(N // BM_N,),
        in_specs=[
            pl.BlockSpec((BM_N, HH), lambda i: (i, 0)),
            pl.BlockSpec((BM_N, HH), lambda i: (i, 0)),
            pl.BlockSpec((BM_N, HH), lambda i: (i, 0)),
            pl.BlockSpec((BM_N, HH), lambda i: (i, 0)),
            pl.BlockSpec((H, H), lambda i: (0, 0)),
            pl.BlockSpec((1, H), lambda i: (0, 0)),
            pl.BlockSpec((H, H), lambda i: (0, 0)),
            pl.BlockSpec((1, H), lambda i: (0, 0)),
            pl.BlockSpec((H, DOUT), lambda i: (0, 0)),
            pl.BlockSpec((1, DOUT), lambda i: (0, 0)),
            pl.BlockSpec((1, 1), lambda i: (0, 0)),
        ],
        out_specs=pl.BlockSpec((1, DOUT), lambda i: (0, 0)),
        out_shape=jax.ShapeDtypeStruct((1, DOUT), _F32),
        scratch_shapes=[pltpu.VMEM((1, H), _F32)],
    )(hlo, hhi, alo, ahi, w1, b1.reshape(1, H), w2, b2.reshape(1, H),
      wsp, bsp.reshape(1, DOUT), prelu_a.reshape(1, 1))


NSUB = 16
TE = E // NSUB
G = 2
GCH = 100
SCH = G * GCH
SER = SCH // EPACK
NSC = TE // SCH
NTILE = 3128
NTILE_LAST = N - 15 * NTILE

_mesh = plsc.VectorSubcoreMesh(core_axis_name="c", subcore_axis_name="s",
                               num_cores=2, num_subcores=NSUB)


@functools.partial(
    pl.kernel,
    out_type=[jax.ShapeDtypeStruct((N, HH), _F32),
              jax.ShapeDtypeStruct((N, HH), _F32)],
    mesh=_mesh,
    scratch_types=[
        pltpu.VMEM_SHARED((N, HH), _F32),
        pltpu.VMEM((2, G, GCH), jnp.int32),
        pltpu.VMEM((2, G, GCH), jnp.int32),
        pltpu.VMEM((SCH, HH), _F32),
        pltpu.VMEM((2, SER, EPACK * HH), _F32),
        pltpu.SemaphoreType.DMA,
        pltpu.SemaphoreType.DMA,
        pltpu.SemaphoreType.DMA,
    ],
    compiler_params=pltpu.CompilerParams(use_tc_tiling_on_sc=False),
)
def _sc_message(hlo, hhi, elo, ehi, src3d, dst3d, olo, ohi,
                aggr, srcb, dstb, rows, ebuf, lsem0, lsem1, gsem):
    c = lax.axis_index("c")
    s = lax.axis_index("s")

    def run(h_ref, e_ref, out_ref):
        def zero_body(j, carry):
            z = jnp.zeros((16,), _F32)
            rows[j, pl.ds(0, 16)] = z
            rows[j, pl.ds(16, 16)] = z
            return carry
        lax.fori_loop(0, SCH, zero_body, 0)
        base = s * NTILE
        nfull = NTILE // SCH
        for k in range(nfull):
            pltpu.sync_copy(rows, aggr.at[pl.ds(base + k * SCH, SCH)])

        @pl.when(s < NSUB - 1)
        def _():
            pltpu.sync_copy(rows.at[pl.ds(0, NTILE - nfull * SCH)],
                            aggr.at[pl.ds(base + nfull * SCH, NTILE - nfull * SCH)])

        @pl.when(s == NSUB - 1)
        def _():
            pltpu.sync_copy(rows.at[pl.ds(0, NTILE_LAST - nfull * SCH)],
                            aggr.at[pl.ds(base + nfull * SCH, NTILE_LAST - nfull * SCH)])

        plsc.subcore_barrier()

        ebase = s * TE
        rbase = s * NSC
        erbase = ebase // EPACK

        def lin_views(t, b):
            return [
                (src3d.at[rbase + t], srcb.at[b]),
                (dst3d.at[rbase + t], dstb.at[b]),
                (e_ref.at[pl.ds(erbase + t * SER, SER)], ebuf.at[b]),
            ]

        def issue_linear(t, b, sem):
            for sv, dv in lin_views(t, b):
                pltpu.async_copy(sv, dv, sem)

        def drain_linear(t, b, sem):
            for sv, dv in lin_views(t, b):
                pltpu.make_async_copy(sv, dv, sem).wait()

        issue_linear(0, 0, lsem0)
        issue_linear(1, 1, lsem1)

        def step(t, b, sem):
            drain_linear(t, b, sem)
            descs = [
                pltpu.async_copy(h_ref.at[srcb.at[b, g]],
                                 rows.at[pl.ds(g * GCH, GCH)], gsem)
                for g in range(G)
            ]
            for d in descs:
                d.wait()

            def addrelu(r, carry):
                for k in range(EPACK):
                    for q in range(2):
                        sl = pl.ds(q * 16, 16)
                        esl = pl.ds(k * HH + q * 16, 16)
                        rows[r * EPACK + k, sl] = jnp.maximum(
                            rows[r * EPACK + k, sl] + ebuf[b, r, esl], 0.0)
                return carry
            lax.fori_loop(0, SER, addrelu, 0)

            for g in range(G):
                pltpu.sync_copy(rows.at[pl.ds(g * GCH, GCH)],
                                aggr.at[dstb.at[b, g]], add=True)

            @pl.when(t + 2 < NSC)
            def _():
                issue_linear(t + 2, b, sem)

        def pair(i, carry):
            step(2 * i, 0, lsem0)
            step(2 * i + 1, 1, lsem1)
            return carry
        lax.fori_loop(0, NSC // 2, pair, 0)

        plsc.subcore_barrier()

        @pl.when(s < NSUB - 1)
        def _():
            pltpu.sync_copy(aggr.at[pl.ds(base, NTILE)],
                            out_ref.at[pl.ds(base, NTILE)])

        @pl.when(s == NSUB - 1)
        def _():
            pltpu.sync_copy(aggr.at[pl.ds(base, NTILE_LAST)],
                            out_ref.at[pl.ds(base, NTILE_LAST)])

    @pl.when(c == 0)
    def _():
        run(hlo, elo, olo)

    @pl.when(c == 1)
    def _():
        run(hhi, ehi, ohi)


def kernel(x, edge_index, edge_attr, Wpn, bpn, Wpe, bpe, W1_0, b1_0, W2_0, b2_0,
           W1_1, b1_1, W2_1, b2_1, W1_2, b1_2, W2_2, b2_2, Wsp, bsp, prelu_a):
    src3d = edge_index[0].reshape(E // SCH, G, GCH)
    dst3d = edge_index[1].reshape(E // SCH, G, GCH)
    hlo, hhi = _proj(x, Wpn, bpn, BM_N, relu=True)
    ea8 = edge_attr.reshape(EROWS, EPACK * DE)
    eye8 = jnp.eye(EPACK, dtype=_F32)
    w8lo = jnp.kron(eye8, Wpe[:, :HH])
    w8hi = jnp.kron(eye8, Wpe[:, HH:])
    b8lo = jnp.tile(bpe[:HH], EPACK).reshape(1, EPACK * HH)
    b8hi = jnp.tile(bpe[HH:], EPACK).reshape(1, EPACK * HH)
    elo, ehi = _proj_e(ea8, w8lo, w8hi, b8lo, b8hi)
    for (w1, b1, w2, b2) in ((W1_0, b1_0, W2_0, b2_0), (W1_1, b1_1, W2_1, b2_1)):
        alo, ahi = _sc_message(hlo, hhi, elo, ehi, src3d, dst3d)
        hlo, hhi = _mlp(hlo, hhi, alo, ahi, w1, b1, w2, b2)
    alo, ahi = _sc_message(hlo, hhi, elo, ehi, src3d, dst3d)
    return _final(hlo, hhi, alo, ahi, W1_2, b1_2, W2_2, b2_2, Wsp, bsp, prelu_a)
